# initial kernel scaffold (unmeasured)
import jax
import jax.numpy as jnp
from jax import lax
from jax.experimental import pallas as pl
from jax.experimental.pallas import tpu as pltpu

_DEVICE_ID_TYPE = getattr(pltpu, "DeviceIdType", None) or pl.DeviceIdType

B, SQ, H, D = 8, 8, 16, 128
SKV = 1024
SCALE = D ** -0.5


def _phase1_body(q_ref, k_ref, v_ref, o_ref, m_ref, l_ref):
    q = q_ref[0, :, 0, :]
    k = k_ref[0, :, 0, :]
    v = v_ref[0, :, 0, :]
    s = lax.dot_general(
        q, k, (((1,), (1,)), ((), ())), preferred_element_type=jnp.float32
    ) * SCALE
    m = jnp.max(s, axis=1, keepdims=True)
    p = jnp.exp(s - m)
    l = jnp.sum(p, axis=1, keepdims=True)
    o = lax.dot_general(
        p, v, (((1,), (0,)), ((), ())), preferred_element_type=jnp.float32
    )
    o_ref[0, 0] = o
    m_ref[0, 0] = jnp.broadcast_to(m, (SQ, D))
    l_ref[0, 0] = jnp.broadcast_to(l, (SQ, D))


def _phase2_body(
    o_ref, m_ref, l_ref, out_ref, ro_ref, rm_ref, rl_ref, send_sems, recv_sems
):
    my_x = lax.axis_index("x")
    my_y = lax.axis_index("y")
    tgt = (1 - my_x, my_y)
    rdmas = []
    for i, (s_ref, d_ref) in enumerate(
        [(o_ref, ro_ref), (m_ref, rm_ref), (l_ref, rl_ref)]
    ):
        c = pltpu.make_async_remote_copy(
            src_ref=s_ref,
            dst_ref=d_ref,
            send_sem=send_sems.at[i],
            recv_sem=recv_sems.at[i],
            device_id=tgt,
            device_id_type=_DEVICE_ID_TYPE.MESH,
        )
        c.start()
        rdmas.append(c)
    for c in rdmas:
        c.wait()

    m1 = m_ref[...]
    l1 = l_ref[...]
    m2 = rm_ref[...]
    l2 = rl_ref[...]
    m = jnp.maximum(m1, m2)
    a1 = jnp.exp(m1 - m)
    a2 = jnp.exp(m2 - m)
    denom = a1 * l1 + a2 * l2
    w1 = (a1 / denom)[..., None]
    w2 = (a2 / denom)[..., None]
    out_ref[...] = o_ref[...] * w1 + ro_ref[...] * w2


def kernel(Q, K, V):
    o_num, m_big, l_big = pl.pallas_call(
        _phase1_body,
        grid=(B, H),
        in_specs=[
            pl.BlockSpec((1, SQ, 1, D), lambda b, h: (b, 0, h, 0)),
            pl.BlockSpec((1, SKV, 1, D), lambda b, h: (b, 0, h, 0)),
            pl.BlockSpec((1, SKV, 1, D), lambda b, h: (b, 0, h, 0)),
        ],
        out_specs=[
            pl.BlockSpec((1, 1, SQ, D), lambda b, h: (b, h, 0, 0)),
            pl.BlockSpec((1, 1, SQ, D), lambda b, h: (b, h, 0, 0)),
            pl.BlockSpec((1, 1, SQ, D), lambda b, h: (b, h, 0, 0)),
        ],
        out_shape=[
            jax.ShapeDtypeStruct((B, H, SQ, D), jnp.float32),
            jax.ShapeDtypeStruct((B, H, SQ, D), jnp.float32),
            jax.ShapeDtypeStruct((B, H, SQ, D), jnp.float32),
        ],
    )(Q, K, V)

    m_small = m_big[:, :, :, 0]
    l_small = l_big[:, :, :, 0]

    out_bhqd = pl.pallas_call(
        _phase2_body,
        in_specs=[
            pl.BlockSpec(memory_space=pltpu.VMEM),
            pl.BlockSpec(memory_space=pltpu.VMEM),
            pl.BlockSpec(memory_space=pltpu.VMEM),
        ],
        out_specs=pl.BlockSpec(memory_space=pltpu.VMEM),
        out_shape=jax.ShapeDtypeStruct((B, H, SQ, D), jnp.float32),
        scratch_shapes=[
            pltpu.VMEM((B, H, SQ, D), jnp.float32),
            pltpu.VMEM((B, H, SQ), jnp.float32),
            pltpu.VMEM((B, H, SQ), jnp.float32),
            pltpu.SemaphoreType.DMA((3,)),
            pltpu.SemaphoreType.DMA((3,)),
        ],
        compiler_params=pltpu.CompilerParams(collective_id=0, has_side_effects=True),
    )(o_num, m_small, l_small)

    return jnp.transpose(out_bhqd, (0, 2, 1, 3))


# baseline (device time: 283607 ns/iter reference)
import jax
import jax.numpy as jnp
from jax import lax
from jax.experimental import pallas as pl
from jax.experimental.pallas import tpu as pltpu

_DEVICE_ID_TYPE = getattr(pltpu, "DeviceIdType", None) or pl.DeviceIdType

B, SQ, H, D = 8, 8, 16, 128
SKV = 1024
SCALE = D ** -0.5


def _phase1_body(q_ref, k_ref, v_ref, o_ref, m_ref, l_ref):
    q = q_ref[0]
    k = k_ref[0]
    v = v_ref[0]
    s = lax.dot_general(
        q, k, (((1,), (1,)), ((), ())), preferred_element_type=jnp.float32
    ) * SCALE
    m = jnp.max(s, axis=1, keepdims=True)
    p = jnp.exp(s - m)
    l = jnp.sum(p, axis=1, keepdims=True)
    o = lax.dot_general(
        p, v, (((1,), (0,)), ((), ())), preferred_element_type=jnp.float32
    )
    o_ref[0, 0] = o
    m_ref[0, 0] = jnp.broadcast_to(m, (SQ, D))
    l_ref[0, 0] = jnp.broadcast_to(l, (SQ, D))


def _phase2_body(
    o_ref, m_ref, l_ref, out_ref, ro_ref, rm_ref, rl_ref, send_sems, recv_sems
):
    my_x = lax.axis_index("x")
    my_y = lax.axis_index("y")
    tgt = (1 - my_x, my_y)
    rdmas = []
    for i, (s_ref, d_ref) in enumerate(
        [(o_ref, ro_ref), (m_ref, rm_ref), (l_ref, rl_ref)]
    ):
        c = pltpu.make_async_remote_copy(
            src_ref=s_ref,
            dst_ref=d_ref,
            send_sem=send_sems.at[i],
            recv_sem=recv_sems.at[i],
            device_id=tgt,
            device_id_type=_DEVICE_ID_TYPE.MESH,
        )
        c.start()
        rdmas.append(c)
    for c in rdmas:
        c.wait()

    m1 = m_ref[...]
    l1 = l_ref[...]
    m2 = rm_ref[...]
    l2 = rl_ref[...]
    m = jnp.maximum(m1, m2)
    a1 = jnp.exp(m1 - m)
    a2 = jnp.exp(m2 - m)
    denom = a1 * l1 + a2 * l2
    w1 = (a1 / denom)[..., None]
    w2 = (a2 / denom)[..., None]
    out_ref[...] = o_ref[...] * w1 + ro_ref[...] * w2


def kernel(Q, K, V):
    Qf = Q.reshape(B, SQ, H * D)
    Kf = K.reshape(B, SKV, H * D)
    Vf = V.reshape(B, SKV, H * D)
    o_num, m_big, l_big = pl.pallas_call(
        _phase1_body,
        grid=(B, H),
        in_specs=[
            pl.BlockSpec((1, SQ, D), lambda b, h: (b, 0, h)),
            pl.BlockSpec((1, SKV, D), lambda b, h: (b, 0, h)),
            pl.BlockSpec((1, SKV, D), lambda b, h: (b, 0, h)),
        ],
        out_specs=[
            pl.BlockSpec((1, 1, SQ, D), lambda b, h: (b, h, 0, 0)),
            pl.BlockSpec((1, 1, SQ, D), lambda b, h: (b, h, 0, 0)),
            pl.BlockSpec((1, 1, SQ, D), lambda b, h: (b, h, 0, 0)),
        ],
        out_shape=[
            jax.ShapeDtypeStruct((B, H, SQ, D), jnp.float32),
            jax.ShapeDtypeStruct((B, H, SQ, D), jnp.float32),
            jax.ShapeDtypeStruct((B, H, SQ, D), jnp.float32),
        ],
    )(Qf, Kf, Vf)

    m_small = m_big[:, :, :, 0]
    l_small = l_big[:, :, :, 0]

    out_bhqd = pl.pallas_call(
        _phase2_body,
        in_specs=[
            pl.BlockSpec(memory_space=pltpu.VMEM),
            pl.BlockSpec(memory_space=pltpu.VMEM),
            pl.BlockSpec(memory_space=pltpu.VMEM),
        ],
        out_specs=pl.BlockSpec(memory_space=pltpu.VMEM),
        out_shape=jax.ShapeDtypeStruct((B, H, SQ, D), jnp.float32),
        scratch_shapes=[
            pltpu.VMEM((B, H, SQ, D), jnp.float32),
            pltpu.VMEM((B, H, SQ), jnp.float32),
            pltpu.VMEM((B, H, SQ), jnp.float32),
            pltpu.SemaphoreType.DMA((3,)),
            pltpu.SemaphoreType.DMA((3,)),
        ],
        compiler_params=pltpu.CompilerParams(has_side_effects=True),
    )(o_num, m_small, l_small)

    return jnp.transpose(out_bhqd, (0, 2, 1, 3))


# device time: 213703 ns/iter; 1.3271x vs baseline; 1.3271x over previous
import jax
import jax.numpy as jnp
from jax import lax
from jax.experimental import pallas as pl
from jax.experimental.pallas import tpu as pltpu

_DEVICE_ID_TYPE = getattr(pltpu, "DeviceIdType", None) or pl.DeviceIdType

B, SQ, H, D = 8, 8, 16, 128
SKV = 1024
SKV_HALF = SKV // 2
SCALE = D ** -0.5


def _phase1_body(y_ref, q_ref, k_ref, v_ref, o_ref, m_ref, l_ref):
    del y_ref
    qs = (q_ref[0] * SCALE).astype(jnp.bfloat16)
    k = k_ref[0, 0].astype(jnp.bfloat16)
    v = v_ref[0, 0].astype(jnp.bfloat16)
    for h in range(H):
        sl = slice(h * D, (h + 1) * D)
        s = lax.dot_general(
            qs[:, sl], k[:, sl], (((1,), (1,)), ((), ())),
            preferred_element_type=jnp.float32,
        )
        m = jnp.max(s, axis=1, keepdims=True)
        p = jnp.exp(s - m)
        l = jnp.sum(p, axis=1, keepdims=True)
        o = lax.dot_general(
            p.astype(jnp.bfloat16), v[:, sl], (((1,), (0,)), ((), ())),
            preferred_element_type=jnp.float32,
        )
        o_ref[0, h] = o
        m_ref[0, h] = jnp.broadcast_to(m, (SQ, D))
        l_ref[0, h] = jnp.broadcast_to(l, (SQ, D))


def _phase2_body(
    o_ref, m_ref, l_ref, out_ref,
    bo_ref, bm_ref, bl_ref,
    ro_ref, rm_ref, rl_ref,
    r2o_ref, r2m_ref, r2l_ref,
    send_sems, recv_sems,
):
    my_x = lax.axis_index("x")
    my_y = lax.axis_index("y")

    def exchange(tgt, pairs, base):
        rdmas = []
        for i, (s_ref, d_ref) in enumerate(pairs):
            c = pltpu.make_async_remote_copy(
                src_ref=s_ref,
                dst_ref=d_ref,
                send_sem=send_sems.at[base + i],
                recv_sem=recv_sems.at[base + i],
                device_id=tgt,
                device_id_type=_DEVICE_ID_TYPE.MESH,
            )
            c.start()
            rdmas.append(c)
        for c in rdmas:
            c.wait()

    exchange(
        (1 - my_x, my_y),
        [(o_ref, ro_ref), (m_ref, rm_ref), (l_ref, rl_ref)],
        0,
    )
    m1 = m_ref[...]
    l1 = l_ref[...]
    m2 = rm_ref[...]
    l2 = rl_ref[...]
    mx = jnp.maximum(m1, m2)
    a1 = jnp.exp(m1 - mx)
    a2 = jnp.exp(m2 - mx)
    bm_ref[...] = mx
    bl_ref[...] = a1 * l1 + a2 * l2
    bo_ref[...] = o_ref[...] * a1[..., None] + ro_ref[...] * a2[..., None]

    exchange(
        (my_x, 1 - my_y),
        [(bo_ref, r2o_ref), (bm_ref, r2m_ref), (bl_ref, r2l_ref)],
        3,
    )
    m1 = bm_ref[...]
    l1 = bl_ref[...]
    m2 = r2m_ref[...]
    l2 = r2l_ref[...]
    mx = jnp.maximum(m1, m2)
    a1 = jnp.exp(m1 - mx)
    a2 = jnp.exp(m2 - mx)
    denom = a1 * l1 + a2 * l2
    w1 = (a1 / denom)[..., None]
    w2 = (a2 / denom)[..., None]
    out_ref[...] = bo_ref[...] * w1 + r2o_ref[...] * w2


def kernel(Q, K, V):
    Qf = Q.reshape(B, SQ, H * D)
    Kf = K.reshape(B, 2, SKV_HALF, H * D)
    Vf = V.reshape(B, 2, SKV_HALF, H * D)
    y_idx = jnp.reshape(lax.axis_index("y"), (1,)).astype(jnp.int32)

    grid_spec = pltpu.PrefetchScalarGridSpec(
        num_scalar_prefetch=1,
        grid=(B,),
        in_specs=[
            pl.BlockSpec((1, SQ, H * D), lambda b, y: (b, 0, 0)),
            pl.BlockSpec((1, 1, SKV_HALF, H * D), lambda b, y: (b, y[0], 0, 0)),
            pl.BlockSpec((1, 1, SKV_HALF, H * D), lambda b, y: (b, y[0], 0, 0)),
        ],
        out_specs=[
            pl.BlockSpec((1, H, SQ, D), lambda b, y: (b, 0, 0, 0)),
            pl.BlockSpec((1, H, SQ, D), lambda b, y: (b, 0, 0, 0)),
            pl.BlockSpec((1, H, SQ, D), lambda b, y: (b, 0, 0, 0)),
        ],
    )
    o_num, m_big, l_big = pl.pallas_call(
        _phase1_body,
        grid_spec=grid_spec,
        out_shape=[
            jax.ShapeDtypeStruct((B, H, SQ, D), jnp.float32),
            jax.ShapeDtypeStruct((B, H, SQ, D), jnp.float32),
            jax.ShapeDtypeStruct((B, H, SQ, D), jnp.float32),
        ],
    )(y_idx, Qf, Kf, Vf)

    m_small = m_big[:, :, :, 0]
    l_small = l_big[:, :, :, 0]

    out_bhqd = pl.pallas_call(
        _phase2_body,
        in_specs=[
            pl.BlockSpec(memory_space=pltpu.VMEM),
            pl.BlockSpec(memory_space=pltpu.VMEM),
            pl.BlockSpec(memory_space=pltpu.VMEM),
        ],
        out_specs=pl.BlockSpec(memory_space=pltpu.VMEM),
        out_shape=jax.ShapeDtypeStruct((B, H, SQ, D), jnp.float32),
        scratch_shapes=[
            pltpu.VMEM((B, H, SQ, D), jnp.float32),
            pltpu.VMEM((B, H, SQ), jnp.float32),
            pltpu.VMEM((B, H, SQ), jnp.float32),
            pltpu.VMEM((B, H, SQ, D), jnp.float32),
            pltpu.VMEM((B, H, SQ), jnp.float32),
            pltpu.VMEM((B, H, SQ), jnp.float32),
            pltpu.VMEM((B, H, SQ, D), jnp.float32),
            pltpu.VMEM((B, H, SQ), jnp.float32),
            pltpu.VMEM((B, H, SQ), jnp.float32),
            pltpu.SemaphoreType.DMA((6,)),
            pltpu.SemaphoreType.DMA((6,)),
        ],
        compiler_params=pltpu.CompilerParams(has_side_effects=True),
    )(o_num, m_small, l_small)

    return jnp.transpose(out_bhqd, (0, 2, 1, 3))


# device time: 123203 ns/iter; 2.3019x vs baseline; 1.7346x over previous
import jax
import jax.numpy as jnp
from jax import lax
from jax.experimental import pallas as pl
from jax.experimental.pallas import tpu as pltpu

_DEVICE_ID_TYPE = getattr(pltpu, "DeviceIdType", None) or pl.DeviceIdType

B, SQ, H, D = 8, 8, 16, 128
SKV = 1024
SKV_HALF = SKV // 2
SCALE = D ** -0.5


def _phase1_body(y_ref, q_ref, k_ref, v_ref, o_ref, m_ref, l_ref):
    del y_ref
    qs = (q_ref[0] * SCALE).astype(jnp.bfloat16)
    k = k_ref[0].astype(jnp.bfloat16)
    v = v_ref[0].astype(jnp.bfloat16)
    s = lax.dot_general(
        qs, k, (((2,), (2,)), ((1,), (1,))),
        preferred_element_type=jnp.float32,
    )
    m = jnp.max(s, axis=2, keepdims=True)
    p = jnp.exp(s - m)
    l = jnp.sum(p, axis=2, keepdims=True)
    o = lax.dot_general(
        p.astype(jnp.bfloat16), v, (((2,), (0,)), ((0,), (1,))),
        preferred_element_type=jnp.float32,
    )
    o_ref[0] = o
    m_ref[0] = jnp.broadcast_to(m, (H, SQ, D))
    l_ref[0] = jnp.broadcast_to(l, (H, SQ, D))


def _phase2_body(
    o_ref, m_ref, l_ref, out_ref,
    bo_ref, bm_ref, bl_ref,
    ro_ref, rm_ref, rl_ref,
    r2o_ref, r2m_ref, r2l_ref,
    send_sems, recv_sems,
):
    my_x = lax.axis_index("x")
    my_y = lax.axis_index("y")

    barrier_sem = pltpu.get_barrier_semaphore()
    for nbr in [(1 - my_x, my_y), (my_x, 1 - my_y)]:
        pl.semaphore_signal(
            barrier_sem, inc=1, device_id=nbr,
            device_id_type=_DEVICE_ID_TYPE.MESH,
        )
    pl.semaphore_wait(barrier_sem, 2)

    def exchange(tgt, pairs, base):
        rdmas = []
        for i, (s_ref, d_ref) in enumerate(pairs):
            c = pltpu.make_async_remote_copy(
                src_ref=s_ref,
                dst_ref=d_ref,
                send_sem=send_sems.at[base + i],
                recv_sem=recv_sems.at[base + i],
                device_id=tgt,
                device_id_type=_DEVICE_ID_TYPE.MESH,
            )
            c.start()
            rdmas.append(c)
        for c in rdmas:
            c.wait()

    exchange(
        (1 - my_x, my_y),
        [(o_ref, ro_ref), (m_ref, rm_ref), (l_ref, rl_ref)],
        0,
    )
    m1 = m_ref[...]
    l1 = l_ref[...]
    m2 = rm_ref[...]
    l2 = rl_ref[...]
    mx = jnp.maximum(m1, m2)
    a1 = jnp.exp(m1 - mx)
    a2 = jnp.exp(m2 - mx)
    bm_ref[...] = mx
    bl_ref[...] = a1 * l1 + a2 * l2
    bo_ref[...] = o_ref[...] * a1[..., None] + ro_ref[...] * a2[..., None]

    exchange(
        (my_x, 1 - my_y),
        [(bo_ref, r2o_ref), (bm_ref, r2m_ref), (bl_ref, r2l_ref)],
        3,
    )
    m1 = bm_ref[...]
    l1 = bl_ref[...]
    m2 = r2m_ref[...]
    l2 = r2l_ref[...]
    mx = jnp.maximum(m1, m2)
    a1 = jnp.exp(m1 - mx)
    a2 = jnp.exp(m2 - mx)
    denom = a1 * l1 + a2 * l2
    w1 = (a1 / denom)[..., None]
    w2 = (a2 / denom)[..., None]
    out_ref[...] = bo_ref[...] * w1 + r2o_ref[...] * w2


def kernel(Q, K, V):
    y_idx = jnp.reshape(lax.axis_index("y"), (1,)).astype(jnp.int32)

    grid_spec = pltpu.PrefetchScalarGridSpec(
        num_scalar_prefetch=1,
        grid=(B,),
        in_specs=[
            pl.BlockSpec((1, SQ, H, D), lambda b, y: (b, 0, 0, 0)),
            pl.BlockSpec((1, SKV_HALF, H, D), lambda b, y: (b, y[0], 0, 0)),
            pl.BlockSpec((1, SKV_HALF, H, D), lambda b, y: (b, y[0], 0, 0)),
        ],
        out_specs=[
            pl.BlockSpec((1, H, SQ, D), lambda b, y: (b, 0, 0, 0)),
            pl.BlockSpec((1, H, SQ, D), lambda b, y: (b, 0, 0, 0)),
            pl.BlockSpec((1, H, SQ, D), lambda b, y: (b, 0, 0, 0)),
        ],
    )
    o_num, m_big, l_big = pl.pallas_call(
        _phase1_body,
        grid_spec=grid_spec,
        out_shape=[
            jax.ShapeDtypeStruct((B, H, SQ, D), jnp.float32),
            jax.ShapeDtypeStruct((B, H, SQ, D), jnp.float32),
            jax.ShapeDtypeStruct((B, H, SQ, D), jnp.float32),
        ],
        compiler_params=pltpu.CompilerParams(
            vmem_limit_bytes=100 * 1024 * 1024
        ),
    )(y_idx, Q, K, V)

    m_small = m_big[:, :, :, 0]
    l_small = l_big[:, :, :, 0]

    out_bhqd = pl.pallas_call(
        _phase2_body,
        in_specs=[
            pl.BlockSpec(memory_space=pltpu.VMEM),
            pl.BlockSpec(memory_space=pltpu.VMEM),
            pl.BlockSpec(memory_space=pltpu.VMEM),
        ],
        out_specs=pl.BlockSpec(memory_space=pltpu.VMEM),
        out_shape=jax.ShapeDtypeStruct((B, H, SQ, D), jnp.float32),
        scratch_shapes=[
            pltpu.VMEM((B, H, SQ, D), jnp.float32),
            pltpu.VMEM((B, H, SQ), jnp.float32),
            pltpu.VMEM((B, H, SQ), jnp.float32),
            pltpu.VMEM((B, H, SQ, D), jnp.float32),
            pltpu.VMEM((B, H, SQ), jnp.float32),
            pltpu.VMEM((B, H, SQ), jnp.float32),
            pltpu.VMEM((B, H, SQ, D), jnp.float32),
            pltpu.VMEM((B, H, SQ), jnp.float32),
            pltpu.VMEM((B, H, SQ), jnp.float32),
            pltpu.SemaphoreType.DMA((6,)),
            pltpu.SemaphoreType.DMA((6,)),
        ],
        compiler_params=pltpu.CompilerParams(
            has_side_effects=True, collective_id=0
        ),
    )(o_num, m_small, l_small)

    return jnp.transpose(out_bhqd, (0, 2, 1, 3))
